# baseline (device time: 36787 ns/iter reference)
import jax
import jax.numpy as jnp
from jax import lax
from jax.experimental import pallas as pl
from jax.experimental.pallas import tpu as pltpu

N_DEV = 8
EPS = 1e-5


def kernel(x, gamma, beta):
    m, n_per = x.shape
    n_global = n_per * N_DEV

    def body(x_ref, g_ref, b_ref, out_ref, comm_ref, send_sems, recv_sems):
        my = lax.axis_index("i")

        xv = x_ref[:, :]
        psum = jnp.sum(xv, axis=1, keepdims=True)
        psq = jnp.sum(xv * xv, axis=1, keepdims=True)
        comm_ref[my] = jnp.concatenate([psum, psq], axis=1)

        rdmas = []
        for d in range(1, N_DEV):
            tgt = lax.rem(my + d, N_DEV)
            rdma = pltpu.make_async_remote_copy(
                src_ref=comm_ref.at[my],
                dst_ref=comm_ref.at[my],
                send_sem=send_sems.at[d - 1],
                recv_sem=recv_sems.at[d - 1],
                device_id=(tgt,),
                device_id_type=pl.DeviceIdType.MESH,
            )
            rdma.start()
            rdmas.append(rdma)

        for d in range(1, N_DEV):
            src = lax.rem(my - d + N_DEV, N_DEV)
            recv = pltpu.make_async_remote_copy(
                src_ref=comm_ref.at[src],
                dst_ref=comm_ref.at[src],
                send_sem=send_sems.at[d - 1],
                recv_sem=recv_sems.at[d - 1],
                device_id=(my,),
                device_id_type=pl.DeviceIdType.MESH,
            )
            recv.wait_recv()

        totals = jnp.sum(comm_ref[:, :, :], axis=0)
        mean = totals[:, 0:1] / n_global
        ex2 = totals[:, 1:2] / n_global
        var = ex2 - mean * mean
        inv = lax.rsqrt(var + EPS)
        out_ref[:, :] = (xv - mean) * inv * g_ref[0, :] + b_ref[0, :]

        for rdma in rdmas:
            rdma.wait_send()

    return pl.pallas_call(
        body,
        out_shape=jax.ShapeDtypeStruct((m, n_per), jnp.float32),
        in_specs=[
            pl.BlockSpec(memory_space=pltpu.VMEM),
            pl.BlockSpec(memory_space=pltpu.VMEM),
            pl.BlockSpec(memory_space=pltpu.VMEM),
        ],
        out_specs=pl.BlockSpec(memory_space=pltpu.VMEM),
        scratch_shapes=[
            pltpu.VMEM((N_DEV, m, 2), jnp.float32),
            pltpu.SemaphoreType.DMA((N_DEV - 1,)),
            pltpu.SemaphoreType.DMA((N_DEV - 1,)),
        ],
    )(x, gamma.reshape(1, n_per), beta.reshape(1, n_per))


# device time: 32794 ns/iter; 1.1218x vs baseline; 1.1218x over previous
import jax
import jax.numpy as jnp
from jax import lax
from jax.experimental import pallas as pl
from jax.experimental.pallas import tpu as pltpu

N_DEV = 8
EPS = 1e-5


def kernel(x, gamma, beta):
    m, n_per = x.shape
    n_global = n_per * N_DEV

    def body(x_ref, g_ref, b_ref, out_ref, comm_ref, send_sems, recv_sems,
             ready_sems):
        my = lax.axis_index("i")

        barrier_sem = pltpu.get_barrier_semaphore()
        pl.semaphore_signal(barrier_sem, inc=1)
        pl.semaphore_wait(barrier_sem, 1)

        with jax.named_scope("announce"):
            for d in range(1, N_DEV):
                tgt = lax.rem(my + d, N_DEV)
                pl.semaphore_signal(
                    ready_sems.at[my],
                    inc=1,
                    device_id=(tgt,),
                    device_id_type=pl.DeviceIdType.MESH,
                )

        with jax.named_scope("partials"):
            xv = x_ref[:, :]
            psum = jnp.sum(xv, axis=1, keepdims=True)
            psq = jnp.sum(xv * xv, axis=1, keepdims=True)
            comm_ref[my] = jnp.concatenate([psum, psq], axis=1)

        rdmas = []
        with jax.named_scope("send"):
            for d in range(1, N_DEV):
                tgt = lax.rem(my + d, N_DEV)
                pl.semaphore_wait(ready_sems.at[tgt], 1)
                rdma = pltpu.make_async_remote_copy(
                    src_ref=comm_ref.at[my],
                    dst_ref=comm_ref.at[my],
                    send_sem=send_sems.at[d - 1],
                    recv_sem=recv_sems.at[d - 1],
                    device_id=(tgt,),
                    device_id_type=pl.DeviceIdType.MESH,
                )
                rdma.start()
                rdmas.append(rdma)

        with jax.named_scope("wait_recv"):
            for d in range(1, N_DEV):
                src = lax.rem(my - d + N_DEV, N_DEV)
                recv = pltpu.make_async_remote_copy(
                    src_ref=comm_ref.at[src],
                    dst_ref=comm_ref.at[src],
                    send_sem=send_sems.at[d - 1],
                    recv_sem=recv_sems.at[d - 1],
                    device_id=(my,),
                    device_id_type=pl.DeviceIdType.MESH,
                )
                recv.wait_recv()

        with jax.named_scope("normalize"):
            totals = jnp.sum(comm_ref[:, :, :], axis=0)
            mean = totals[:, 0:1] / n_global
            ex2 = totals[:, 1:2] / n_global
            var = ex2 - mean * mean
            inv = lax.rsqrt(var + EPS)
            out_ref[:, :] = (xv - mean) * inv * g_ref[0, :] + b_ref[0, :]

        with jax.named_scope("drain"):
            for rdma in rdmas:
                rdma.wait_send()

    return pl.pallas_call(
        body,
        out_shape=jax.ShapeDtypeStruct((m, n_per), jnp.float32),
        in_specs=[
            pl.BlockSpec(memory_space=pltpu.VMEM),
            pl.BlockSpec(memory_space=pltpu.VMEM),
            pl.BlockSpec(memory_space=pltpu.VMEM),
        ],
        out_specs=pl.BlockSpec(memory_space=pltpu.VMEM),
        scratch_shapes=[
            pltpu.VMEM((N_DEV, m, 2), jnp.float32),
            pltpu.SemaphoreType.DMA((N_DEV - 1,)),
            pltpu.SemaphoreType.DMA((N_DEV - 1,)),
            pltpu.SemaphoreType.REGULAR((N_DEV,)),
        ],
        compiler_params=pltpu.CompilerParams(collective_id=0),
    )(x, gamma.reshape(1, n_per), beta.reshape(1, n_per))


# device time: 32749 ns/iter; 1.1233x vs baseline; 1.0014x over previous
import jax
import jax.numpy as jnp
from jax import lax
from jax.experimental import pallas as pl
from jax.experimental.pallas import tpu as pltpu

N_DEV = 8
EPS = 1e-5


def kernel(x, gamma, beta):
    m, n_per = x.shape
    n_global = n_per * N_DEV

    def body(x_ref, g_ref, b_ref, out_ref, local_ref, comm_ref, send_sems,
             recv_sems, ready_sems):
        my = lax.axis_index("i")

        barrier_sem = pltpu.get_barrier_semaphore()
        pl.semaphore_signal(barrier_sem, inc=1)
        pl.semaphore_wait(barrier_sem, 1)

        with jax.named_scope("announce"):
            for d in range(1, N_DEV):
                src = lax.rem(my - d + N_DEV, N_DEV)
                pl.semaphore_signal(
                    ready_sems.at[d - 1],
                    inc=1,
                    device_id=(src,),
                    device_id_type=pl.DeviceIdType.MESH,
                )

        with jax.named_scope("partials"):
            xv = x_ref[:, :]
            psum = jnp.sum(xv, axis=1, keepdims=True)
            psq = jnp.sum(xv * xv, axis=1, keepdims=True)
            local_ref[:, :] = jnp.concatenate([psum, psq], axis=1)

        rdmas = []
        with jax.named_scope("send"):
            for d in range(1, N_DEV):
                tgt = lax.rem(my + d, N_DEV)
                pl.semaphore_wait(ready_sems.at[d - 1], 1)
                rdma = pltpu.make_async_remote_copy(
                    src_ref=local_ref,
                    dst_ref=comm_ref.at[d - 1],
                    send_sem=send_sems.at[d - 1],
                    recv_sem=recv_sems.at[d - 1],
                    device_id=(tgt,),
                    device_id_type=pl.DeviceIdType.MESH,
                )
                rdma.start()
                rdmas.append(rdma)

        with jax.named_scope("wait_recv"):
            for d in range(1, N_DEV):
                recv = pltpu.make_async_remote_copy(
                    src_ref=local_ref,
                    dst_ref=comm_ref.at[d - 1],
                    send_sem=send_sems.at[d - 1],
                    recv_sem=recv_sems.at[d - 1],
                    device_id=(my,),
                    device_id_type=pl.DeviceIdType.MESH,
                )
                recv.wait_recv()

        with jax.named_scope("normalize"):
            totals = local_ref[:, :] + jnp.sum(comm_ref[:, :, :], axis=0)
            mean = totals[:, 0:1] / n_global
            ex2 = totals[:, 1:2] / n_global
            var = ex2 - mean * mean
            inv = lax.rsqrt(var + EPS)
            out_ref[:, :] = (xv - mean) * inv * g_ref[0, :] + b_ref[0, :]

        with jax.named_scope("drain"):
            for rdma in rdmas:
                rdma.wait_send()

    return pl.pallas_call(
        body,
        out_shape=jax.ShapeDtypeStruct((m, n_per), jnp.float32),
        in_specs=[
            pl.BlockSpec(memory_space=pltpu.VMEM),
            pl.BlockSpec(memory_space=pltpu.VMEM),
            pl.BlockSpec(memory_space=pltpu.VMEM),
        ],
        out_specs=pl.BlockSpec(memory_space=pltpu.VMEM),
        scratch_shapes=[
            pltpu.VMEM((m, 2), jnp.float32),
            pltpu.VMEM((N_DEV - 1, m, 2), jnp.float32),
            pltpu.SemaphoreType.DMA((N_DEV - 1,)),
            pltpu.SemaphoreType.DMA((N_DEV - 1,)),
            pltpu.SemaphoreType.REGULAR((N_DEV - 1,)),
        ],
        compiler_params=pltpu.CompilerParams(collective_id=0),
    )(x, gamma.reshape(1, n_per), beta.reshape(1, n_per))


# device time: 11100 ns/iter; 3.3141x vs baseline; 2.9504x over previous
import jax
import jax.numpy as jnp
from jax import lax
from jax.experimental import pallas as pl
from jax.experimental.pallas import tpu as pltpu

N_DEV = 8
EPS = 1e-5


def kernel(x, gamma, beta):
    m, n_per = x.shape
    n_global = n_per * N_DEV

    def body(x_ref, g_ref, b_ref, out_ref, local_ref, comm_ref, send_sems,
             recv_sems, ready_sems):
        my = lax.axis_index("i")

        barrier_sem = pltpu.get_barrier_semaphore()
        pl.semaphore_signal(barrier_sem, inc=1)
        pl.semaphore_wait(barrier_sem, 1)

        with jax.named_scope("announce"):
            for d in range(1, N_DEV):
                src = lax.rem(my - d + N_DEV, N_DEV)
                pl.semaphore_signal(
                    ready_sems.at[d - 1],
                    inc=1,
                    device_id=(src,),
                    device_id_type=pl.DeviceIdType.MESH,
                )

        with jax.named_scope("partials"):
            xv = x_ref[:, :]
            psum = jnp.sum(xv, axis=1)
            psq = jnp.sum(xv * xv, axis=1)
            local_ref[:, :] = jnp.stack([psum, psq])

        rdmas = []
        with jax.named_scope("send"):
            for d in range(1, N_DEV):
                tgt = lax.rem(my + d, N_DEV)
                pl.semaphore_wait(ready_sems.at[d - 1], 1)
                rdma = pltpu.make_async_remote_copy(
                    src_ref=local_ref,
                    dst_ref=comm_ref.at[d - 1],
                    send_sem=send_sems.at[d - 1],
                    recv_sem=recv_sems.at[d - 1],
                    device_id=(tgt,),
                    device_id_type=pl.DeviceIdType.MESH,
                )
                rdma.start()
                rdmas.append(rdma)

        with jax.named_scope("wait_recv"):
            for d in range(1, N_DEV):
                recv = pltpu.make_async_remote_copy(
                    src_ref=local_ref,
                    dst_ref=comm_ref.at[d - 1],
                    send_sem=send_sems.at[d - 1],
                    recv_sem=recv_sems.at[d - 1],
                    device_id=(my,),
                    device_id_type=pl.DeviceIdType.MESH,
                )
                recv.wait_recv()

        with jax.named_scope("normalize"):
            totals = local_ref[:, :] + jnp.sum(comm_ref[:, :, :], axis=0)
            mean_row = totals[0:1, :] / n_global
            ex2_row = totals[1:2, :] / n_global
            var_row = ex2_row - mean_row * mean_row
            inv_row = lax.rsqrt(var_row + EPS)
            mi = jnp.transpose(jnp.concatenate([mean_row, inv_row], axis=0))
            mean = mi[:, 0:1]
            inv = mi[:, 1:2]
            out_ref[:, :] = (xv - mean) * inv * g_ref[0, :] + b_ref[0, :]

        with jax.named_scope("drain"):
            for rdma in rdmas:
                rdma.wait_send()

    return pl.pallas_call(
        body,
        out_shape=jax.ShapeDtypeStruct((m, n_per), jnp.float32),
        in_specs=[
            pl.BlockSpec(memory_space=pltpu.VMEM),
            pl.BlockSpec(memory_space=pltpu.VMEM),
            pl.BlockSpec(memory_space=pltpu.VMEM),
        ],
        out_specs=pl.BlockSpec(memory_space=pltpu.VMEM),
        scratch_shapes=[
            pltpu.VMEM((2, m), jnp.float32),
            pltpu.VMEM((N_DEV - 1, 2, m), jnp.float32),
            pltpu.SemaphoreType.DMA((N_DEV - 1,)),
            pltpu.SemaphoreType.DMA((N_DEV - 1,)),
            pltpu.SemaphoreType.REGULAR((N_DEV - 1,)),
        ],
        compiler_params=pltpu.CompilerParams(collective_id=0),
    )(x, gamma.reshape(1, n_per), beta.reshape(1, n_per))
